# speedup vs baseline: 2.5395x; 2.5395x over previous
import functools

import jax
import jax.numpy as jnp
from jax.experimental import pallas as pl
from jax.experimental.pallas import tpu as pltpu

N = 10000
D = 128
C = 2
G = 16
ROWS = 1000


def _pool_body(x_ref, ids_ref, wf_ref, bf_ref, out_ref, scr_ref):
    i = pl.program_id(0)

    @pl.when(i == 0)
    def _():
        scr_ref[...] = jnp.full((G, D), -jnp.inf, jnp.float32)

    x = x_ref[...]
    ids = ids_ref[...]
    cur = scr_ref[...]
    maxes = jnp.stack(
        [jnp.max(jnp.where(ids == float(g), x, -jnp.inf), axis=0)
         for g in range(G)])
    cur = jnp.maximum(cur, maxes)
    scr_ref[...] = cur

    @pl.when(i == pl.num_programs(0) - 1)
    def _():
        pooled = jnp.where(jnp.isfinite(cur), cur, 0.0)
        out_ref[...] = (
            jnp.dot(pooled, wf_ref[...], preferred_element_type=jnp.float32)
            + bf_ref[...])


def _pool(post, batch, Wf, bf):
    ids = batch.astype(jnp.float32).reshape(N, 1)
    return pl.pallas_call(
        _pool_body,
        grid=(N // ROWS,),
        in_specs=[
            pl.BlockSpec((ROWS, D), lambda i: (i, 0)),
            pl.BlockSpec((ROWS, 1), lambda i: (i, 0)),
            pl.BlockSpec((D, C), lambda i: (0, 0)),
            pl.BlockSpec((1, C), lambda i: (0, 0)),
        ],
        out_specs=pl.BlockSpec((G, C), lambda i: (0, 0)),
        out_shape=jax.ShapeDtypeStruct((G, C), jnp.float32),
        scratch_shapes=[pltpu.VMEM((G, D), jnp.float32)],
    )(post, ids, Wf, bf.reshape(1, C))


def kernel(x, edge_index, batch, W1, b1, W2, b2, W3, b3, Wf, bf):
    src, dst = edge_index[0], edge_index[1]
    deg = jnp.zeros((N,), jnp.float32).at[dst].add(1.0) + 1.0
    dinv = jax.lax.rsqrt(deg)[:, None]

    def layer(a, W, b):
        g = (a @ W) * dinv
        agg = jnp.zeros_like(g).at[dst].add(g[src])
        return dinv * (agg + g) + b

    a = jax.nn.relu(layer(x, W1, b1))
    a = jax.nn.relu(layer(a, W2, b2))
    post = layer(a, W3, b3)
    return _pool(post, batch, Wf, bf)


# speedup vs baseline: 7.1231x; 2.8049x over previous
import functools

import jax
import jax.numpy as jnp
from jax import lax
from jax.experimental import pallas as pl
from jax.experimental.pallas import tpu as pltpu
from jax.experimental.pallas import tpu_sc as plsc

N = 10000
E = 320000
D = 128
C = 2
G = 16

NC = 2
NSUB = 16
NW = NC * NSUB
CH = 128
CHW = (E + NW * CH - 1) // (NW * CH)
CHW = 80
EPAD = NW * CHW * CH
NP = 10240
NZCH = NP // NSUB // CH
DUMMY = N

ROWS = 1000


def _sc_mesh():
    return plsc.VectorSubcoreMesh(core_axis_name="c", subcore_axis_name="s")


@functools.partial(
    pl.kernel,
    out_type=jax.ShapeDtypeStruct((NC, NP, D), jnp.float32),
    mesh=_sc_mesh(),
    scratch_types=[
        pltpu.VMEM((CHW, CH), jnp.int32),
        pltpu.VMEM((CH, D), jnp.float32),
        pltpu.VMEM((CH, D), jnp.float32),
        pltpu.VMEM_SHARED((NP, D), jnp.float32),
    ],
)
def _sc_deg(dstp_hbm, ones_hbm, zeros_hbm, out_hbm, dstw, ones_v, z_v, acc):
    c = lax.axis_index("c")
    s = lax.axis_index("s")
    wid = c * NSUB + s
    pltpu.sync_copy(dstp_hbm.at[wid], dstw)
    pltpu.sync_copy(ones_hbm, ones_v)
    pltpu.sync_copy(zeros_hbm, z_v)
    for kk in range(NZCH):
        pltpu.sync_copy(z_v, acc.at[pl.ds(s * (NP // NSUB) + kk * CH, CH)])
    plsc.subcore_barrier()

    def chunk(j, carry):
        pltpu.sync_copy(ones_v, acc.at[dstw.at[j]], add=True)
        return carry

    lax.fori_loop(0, CHW, chunk, 0)
    plsc.subcore_barrier()
    for kk in range(NZCH):
        off = s * (NP // NSUB) + kk * CH
        pltpu.sync_copy(acc.at[pl.ds(off, CH)], z_v)
        pltpu.sync_copy(z_v, out_hbm.at[c, pl.ds(off, CH)])


@functools.partial(
    pl.kernel,
    out_type=jax.ShapeDtypeStruct((NC, NP, D), jnp.float32),
    mesh=_sc_mesh(),
    scratch_types=[
        pltpu.VMEM((CHW, CH), jnp.int32),
        pltpu.VMEM((CHW, CH), jnp.int32),
        pltpu.VMEM((CH, D), jnp.float32),
        pltpu.VMEM_SHARED((NP, D), jnp.float32),
    ],
)
def _sc_agg(g_hbm, srcp_hbm, dstp_hbm, zrows_hbm, out_hbm,
            srcw, dstw, rows, acc):
    c = lax.axis_index("c")
    s = lax.axis_index("s")
    wid = c * NSUB + s
    pltpu.sync_copy(srcp_hbm.at[wid], srcw)
    pltpu.sync_copy(dstp_hbm.at[wid], dstw)
    pltpu.sync_copy(zrows_hbm, rows)
    for kk in range(NZCH):
        pltpu.sync_copy(rows, acc.at[pl.ds(s * (NP // NSUB) + kk * CH, CH)])
    plsc.subcore_barrier()

    def chunk(j, carry):
        pltpu.sync_copy(g_hbm.at[srcw.at[j]], rows)
        pltpu.sync_copy(rows, acc.at[dstw.at[j]], add=True)
        return carry

    lax.fori_loop(0, CHW, chunk, 0)
    plsc.subcore_barrier()
    for kk in range(NZCH):
        off = s * (NP // NSUB) + kk * CH
        pltpu.sync_copy(acc.at[pl.ds(off, CH)], rows)
        pltpu.sync_copy(rows, out_hbm.at[c, pl.ds(off, CH)])


def _pool_body(x_ref, ids_ref, wf_ref, bf_ref, out_ref, scr_ref):
    i = pl.program_id(0)

    @pl.when(i == 0)
    def _():
        scr_ref[...] = jnp.full((G, D), -jnp.inf, jnp.float32)

    x = x_ref[...]
    ids = ids_ref[...]
    cur = scr_ref[...]
    maxes = jnp.stack(
        [jnp.max(jnp.where(ids == float(g), x, -jnp.inf), axis=0)
         for g in range(G)])
    cur = jnp.maximum(cur, maxes)
    scr_ref[...] = cur

    @pl.when(i == pl.num_programs(0) - 1)
    def _():
        pooled = jnp.where(jnp.isfinite(cur), cur, 0.0)
        out_ref[...] = (
            jnp.dot(pooled, wf_ref[...], preferred_element_type=jnp.float32)
            + bf_ref[...])


def _pool(post, batch, Wf, bf):
    ids = batch.astype(jnp.float32).reshape(N, 1)
    return pl.pallas_call(
        _pool_body,
        grid=(N // ROWS,),
        in_specs=[
            pl.BlockSpec((ROWS, D), lambda i: (i, 0)),
            pl.BlockSpec((ROWS, 1), lambda i: (i, 0)),
            pl.BlockSpec((D, C), lambda i: (0, 0)),
            pl.BlockSpec((1, C), lambda i: (0, 0)),
        ],
        out_specs=pl.BlockSpec((G, C), lambda i: (0, 0)),
        out_shape=jax.ShapeDtypeStruct((G, C), jnp.float32),
        scratch_shapes=[pltpu.VMEM((G, D), jnp.float32)],
    )(post, ids, Wf, bf.reshape(1, C))


def kernel(x, edge_index, batch, W1, b1, W2, b2, W3, b3, Wf, bf):
    src, dst = edge_index[0], edge_index[1]
    pad = EPAD - E
    srcp = jnp.concatenate(
        [src, jnp.zeros((pad,), jnp.int32)]).reshape(NW, CHW, CH)
    dstp = jnp.concatenate(
        [dst, jnp.full((pad,), DUMMY, jnp.int32)]).reshape(NW, CHW, CH)

    ones_rows = jnp.ones((CH, D), jnp.float32)
    zrows = jnp.zeros((CH, D), jnp.float32)

    degp = _sc_deg(dstp, ones_rows, zrows)
    deg = degp[0, :N, 0] + degp[1, :N, 0] + 1.0
    dinv = lax.rsqrt(deg)[:, None]

    def layer(a, W, b):
        g = (a @ W) * dinv
        aggp = _sc_agg(g, srcp, dstp, zrows)
        return dinv * (aggp[0, :N] + aggp[1, :N] + g) + b

    a = jax.nn.relu(layer(x, W1, b1))
    a = jax.nn.relu(layer(a, W2, b2))
    post = layer(a, W3, b3)
    return _pool(post, batch, Wf, bf)
